# Initial kernel scaffold; baseline (speedup 1.0000x reference)
import jax
import jax.numpy as jnp
from jax.experimental import pallas as pl


def kernel(x_user, x_item, W_self_user, W_self_item, W_nbr_u2i, W_nbr_i2u, ln_g_user, ln_b_user, ln_g_item, ln_b_item, edge_index_user_to_item, edge_index_item_to_user):
    raise NotImplementedError("write your pallas kernel here")


# speedup vs baseline: 3.5694x; 3.5694x over previous
import functools

import jax
import jax.numpy as jnp
from jax import lax
from jax.experimental import pallas as pl
from jax.experimental.pallas import tpu as pltpu
from jax.experimental.pallas import tpu_sc as plsc

N = 10000
C = 128
E = 320000
L = 4

NC = 2
NS = 16
LANES = 16

EPT = E // NS
K = 80
NCH = EPT // K
RPT = 640
NP = NS * RPT
CCH = C // LANES

_mesh = plsc.VectorSubcoreMesh(core_axis_name="c", subcore_axis_name="s")


@functools.partial(
    pl.kernel,
    out_type=(
        jax.ShapeDtypeStruct((NP, C), jnp.float32),
        jax.ShapeDtypeStruct((NP, C), jnp.float32),
    ),
    mesh=_mesh,
    compiler_params=pltpu.CompilerParams(use_tc_tiling_on_sc=False),
    scratch_types=dict(
        sidx_v=pltpu.VMEM((K,), jnp.int32),
        didx_v=pltpu.VMEM((K,), jnp.int32),
        rows_v=pltpu.VMEM((K, C), jnp.float32),
        ones_v=pltpu.VMEM((K, LANES), jnp.float32),
        stage_v=pltpu.VMEM((8, C), jnp.float32),
        cntl_v=pltpu.VMEM((8, LANES), jnp.float32),
        acc_sp=pltpu.VMEM_SHARED((NP, C), jnp.float32),
        cnt_sp=pltpu.VMEM_SHARED((NP, LANES), jnp.float32),
    ),
)
def _sc_aggregate(x_user, x_item, su2i, du2i, si2u, di2u,
                  out_i, out_u,
                  sidx_v, didx_v, rows_v, ones_v, stage_v, cntl_v,
                  acc_sp, cnt_sp):
    c = lax.axis_index("c")
    s = lax.axis_index("s")

    zeros16 = jnp.zeros((LANES,), jnp.float32)
    ones16 = jnp.ones((LANES,), jnp.float32)

    def _fill_ones(k, _):
        ones_v[k, :] = ones16
        return _
    lax.fori_loop(0, K, _fill_ones, 0)

    for r in range(8):
        cntl_v[r, :] = zeros16
        for j in range(CCH):
            stage_v[r, pl.ds(j * LANES, LANES)] = zeros16

    base_r = s * RPT
    def _zero_slice(q, _):
        pltpu.sync_copy(stage_v, acc_sp.at[pl.ds(base_r + q * 8, 8)])
        pltpu.sync_copy(cntl_v, cnt_sp.at[pl.ds(base_r + q * 8, 8)])
        return _
    lax.fori_loop(0, RPT // 8, _zero_slice, 0)
    plsc.subcore_barrier()

    def _accumulate(x_hbm, src_hbm, dst_hbm):
        def body(j, _):
            off = s * EPT + j * K
            pltpu.sync_copy(src_hbm.at[pl.ds(off, K)], sidx_v)
            pltpu.sync_copy(dst_hbm.at[pl.ds(off, K)], didx_v)
            pltpu.sync_copy(x_hbm.at[sidx_v], rows_v)
            pltpu.sync_copy(rows_v, acc_sp.at[didx_v], add=True)
            pltpu.sync_copy(ones_v, cnt_sp.at[didx_v], add=True)
            return _
        lax.fori_loop(0, NCH, body, 0)

    @pl.when(c == 0)
    def _():
        _accumulate(x_user, su2i, du2i)

    @pl.when(c == 1)
    def _():
        _accumulate(x_item, si2u, di2u)

    plsc.subcore_barrier()

    def _mean_out(out_hbm):
        def chunk(q, _):
            row0 = base_r + q * 8
            pltpu.sync_copy(acc_sp.at[pl.ds(row0, 8)], stage_v)
            pltpu.sync_copy(cnt_sp.at[pl.ds(row0, 8)], cntl_v)
            for r in range(8):
                cnt = cntl_v[r, :]
                recip = ones16 / jnp.maximum(cnt, ones16)
                for j in range(CCH):
                    v = stage_v[r, pl.ds(j * LANES, LANES)]
                    stage_v[r, pl.ds(j * LANES, LANES)] = v * recip
            pltpu.sync_copy(stage_v, out_hbm.at[pl.ds(row0, 8)])
            return _
        lax.fori_loop(0, RPT // 8, chunk, 0)

    @pl.when(c == 0)
    def _():
        _mean_out(out_i)

    @pl.when(c == 1)
    def _():
        _mean_out(out_u)


BN = 2000


def _tc_body(x_ref, agg_ref, ws_ref, wn_ref, g_ref, b_ref, o_ref):
    h = jnp.dot(x_ref[...], ws_ref[...], preferred_element_type=jnp.float32)
    h = h + jnp.dot(agg_ref[...], wn_ref[...], preferred_element_type=jnp.float32)
    mu = jnp.mean(h, axis=1, keepdims=True)
    var = jnp.mean((h - mu) ** 2, axis=1, keepdims=True)
    hn = (h - mu) * lax.rsqrt(var + 1e-5) * g_ref[...] + b_ref[...]
    o_ref[...] = jnp.maximum(hn, 0.0)


def _tc_dense(x, agg, w_self, w_nbr, g, b):
    return pl.pallas_call(
        _tc_body,
        grid=(N // BN,),
        in_specs=[
            pl.BlockSpec((BN, C), lambda i: (i, 0)),
            pl.BlockSpec((BN, C), lambda i: (i, 0)),
            pl.BlockSpec((C, C), lambda i: (0, 0)),
            pl.BlockSpec((C, C), lambda i: (0, 0)),
            pl.BlockSpec((1, C), lambda i: (0, 0)),
            pl.BlockSpec((1, C), lambda i: (0, 0)),
        ],
        out_specs=pl.BlockSpec((BN, C), lambda i: (i, 0)),
        out_shape=jax.ShapeDtypeStruct((N, C), jnp.float32),
    )(x, agg, w_self, w_nbr, g.reshape(1, C), b.reshape(1, C))


def kernel(x_user, x_item, W_self_user, W_self_item, W_nbr_u2i, W_nbr_i2u,
           ln_g_user, ln_b_user, ln_g_item, ln_b_item,
           edge_index_user_to_item, edge_index_item_to_user):
    su2i = edge_index_user_to_item[0].astype(jnp.int32)
    du2i = edge_index_user_to_item[1].astype(jnp.int32)
    si2u = edge_index_item_to_user[0].astype(jnp.int32)
    di2u = edge_index_item_to_user[1].astype(jnp.int32)

    xu, xi = x_user, x_item
    for l in range(L):
        agg_i, agg_u = _sc_aggregate(xu, xi, su2i, du2i, si2u, di2u)
        xu = _tc_dense(xu, agg_u, W_self_user[l], W_nbr_i2u[l],
                       ln_g_user[l], ln_b_user[l])
        xi = _tc_dense(xi, agg_i, W_self_item[l], W_nbr_u2i[l],
                       ln_g_item[l], ln_b_item[l])
    return xu


# speedup vs baseline: 7.1377x; 1.9997x over previous
import functools

import jax
import jax.numpy as jnp
from jax import lax
from jax.experimental import pallas as pl
from jax.experimental.pallas import tpu as pltpu
from jax.experimental.pallas import tpu_sc as plsc

N = 10000
C = 128
E = 320000
L = 4

NS = 16
LANES = 16

EPT = E // NS
K = 80
NCH = EPT // K
G = 50
NG = NCH // G
RPT = 640
NP = NS * RPT
CCH = C // LANES
ZCH = RPT // K

_mesh = plsc.VectorSubcoreMesh(core_axis_name="c", subcore_axis_name="s")
_params = pltpu.CompilerParams(use_tc_tiling_on_sc=False)


@functools.partial(
    pl.kernel,
    out_type=(
        jax.ShapeDtypeStruct((NP, LANES), jnp.float32),
        jax.ShapeDtypeStruct((NP, LANES), jnp.float32),
    ),
    mesh=_mesh,
    compiler_params=_params,
    scratch_types=dict(
        didx_g=pltpu.VMEM((G, K), jnp.int32),
        ones_v=pltpu.VMEM((K, LANES), jnp.float32),
        zero_v=pltpu.VMEM((K, LANES), jnp.float32),
        cnt_sp=pltpu.VMEM_SHARED((NP, LANES), jnp.float32),
    ),
)
def _sc_counts(du2i, di2u, cnt_i, cnt_u, didx_g, ones_v, zero_v, cnt_sp):
    c = lax.axis_index("c")
    s = lax.axis_index("s")

    zeros16 = jnp.zeros((LANES,), jnp.float32)
    ones16 = jnp.ones((LANES,), jnp.float32)

    def _fill(k, _):
        ones_v[k, :] = ones16
        zero_v[k, :] = zeros16
        return _
    lax.fori_loop(0, K, _fill, 0)

    base_r = s * RPT
    for z in range(ZCH):
        pltpu.sync_copy(zero_v, cnt_sp.at[pl.ds(base_r + z * K, K)])
    plsc.subcore_barrier()

    def _count(dst_hbm):
        def group(g, _):
            pltpu.sync_copy(dst_hbm.at[s, pl.ds(g * G, G)], didx_g)
            for jj in range(G):
                pltpu.sync_copy(ones_v, cnt_sp.at[didx_g.at[jj]], add=True)
            return _
        lax.fori_loop(0, NG, group, 0)

    @pl.when(c == 0)
    def _():
        _count(du2i)

    @pl.when(c == 1)
    def _():
        _count(di2u)

    plsc.subcore_barrier()

    @pl.when(c == 0)
    def _():
        pltpu.sync_copy(cnt_sp.at[pl.ds(base_r, RPT)], cnt_i.at[pl.ds(base_r, RPT)])

    @pl.when(c == 1)
    def _():
        pltpu.sync_copy(cnt_sp.at[pl.ds(base_r, RPT)], cnt_u.at[pl.ds(base_r, RPT)])


@functools.partial(
    pl.kernel,
    out_type=(
        jax.ShapeDtypeStruct((NP, C), jnp.float32),
        jax.ShapeDtypeStruct((NP, C), jnp.float32),
    ),
    mesh=_mesh,
    compiler_params=_params,
    scratch_types=dict(
        sidx_g=pltpu.VMEM((G, K), jnp.int32),
        didx_g=pltpu.VMEM((G, K), jnp.int32),
        rows_a=pltpu.VMEM((K, C), jnp.float32),
        rows_b=pltpu.VMEM((K, C), jnp.float32),
        sem_g=pltpu.SemaphoreType.DMA,
        sem_s=pltpu.SemaphoreType.DMA,
        acc_sp=pltpu.VMEM_SHARED((NP, C), jnp.float32),
    ),
)
def _sc_aggregate(x_user, x_item, su2i, du2i, si2u, di2u,
                  out_i, out_u,
                  sidx_g, didx_g, rows_a, rows_b, sem_g, sem_s, acc_sp):
    c = lax.axis_index("c")
    s = lax.axis_index("s")

    zeros16 = jnp.zeros((LANES,), jnp.float32)

    def _zrow(r, _):
        for j in range(CCH):
            rows_a[r, pl.ds(j * LANES, LANES)] = zeros16
        return _
    lax.fori_loop(0, K, _zrow, 0)

    base_r = s * RPT
    for z in range(ZCH):
        pltpu.sync_copy(rows_a, acc_sp.at[pl.ds(base_r + z * K, K)])
    plsc.subcore_barrier()

    def _wait_gather():
        pltpu.make_async_copy(x_user.at[pl.ds(0, K)], rows_a, sem_g).wait()

    def _wait_scatter():
        pltpu.make_async_copy(x_user.at[pl.ds(0, K)], rows_a, sem_s).wait()

    def _accumulate(x_hbm, src_hbm, dst_hbm):
        rbufs = (rows_a, rows_b)

        def group(g, _):
            pltpu.sync_copy(src_hbm.at[s, pl.ds(g * G, G)], sidx_g)
            pltpu.sync_copy(dst_hbm.at[s, pl.ds(g * G, G)], didx_g)
            pltpu.async_copy(x_hbm.at[sidx_g.at[0]], rbufs[0], sem_g)
            for jj in range(G):
                rp = rbufs[jj % 2]
                _wait_gather()
                if jj + 1 < G:
                    if jj >= 1:
                        _wait_scatter()
                    pltpu.async_copy(x_hbm.at[sidx_g.at[jj + 1]],
                                     rbufs[(jj + 1) % 2], sem_g)
                pltpu.async_copy(rp, acc_sp.at[didx_g.at[jj]], sem_s, add=True)
            _wait_scatter()
            _wait_scatter()
            return _
        lax.fori_loop(0, NG, group, 0)

    @pl.when(c == 0)
    def _():
        _accumulate(x_user, su2i, du2i)

    @pl.when(c == 1)
    def _():
        _accumulate(x_item, si2u, di2u)

    plsc.subcore_barrier()

    @pl.when(c == 0)
    def _():
        pltpu.sync_copy(acc_sp.at[pl.ds(base_r, RPT)], out_i.at[pl.ds(base_r, RPT)])

    @pl.when(c == 1)
    def _():
        pltpu.sync_copy(acc_sp.at[pl.ds(base_r, RPT)], out_u.at[pl.ds(base_r, RPT)])


BN = 2000


def _tc_body(x_ref, agg_ref, cnt_ref, ws_ref, wn_ref, g_ref, b_ref, o_ref):
    recip = 1.0 / jnp.maximum(cnt_ref[...][:, 0:1], 1.0)
    h = jnp.dot(x_ref[...], ws_ref[...], preferred_element_type=jnp.float32)
    h = h + jnp.dot(agg_ref[...] * recip, wn_ref[...],
                    preferred_element_type=jnp.float32)
    mu = jnp.mean(h, axis=1, keepdims=True)
    var = jnp.mean((h - mu) ** 2, axis=1, keepdims=True)
    hn = (h - mu) * lax.rsqrt(var + 1e-5) * g_ref[...] + b_ref[...]
    o_ref[...] = jnp.maximum(hn, 0.0)


def _tc_dense(x, agg, cnt, w_self, w_nbr, g, b):
    return pl.pallas_call(
        _tc_body,
        grid=(N // BN,),
        in_specs=[
            pl.BlockSpec((BN, C), lambda i: (i, 0)),
            pl.BlockSpec((BN, C), lambda i: (i, 0)),
            pl.BlockSpec((BN, LANES), lambda i: (i, 0)),
            pl.BlockSpec((C, C), lambda i: (0, 0)),
            pl.BlockSpec((C, C), lambda i: (0, 0)),
            pl.BlockSpec((1, C), lambda i: (0, 0)),
            pl.BlockSpec((1, C), lambda i: (0, 0)),
        ],
        out_specs=pl.BlockSpec((BN, C), lambda i: (i, 0)),
        out_shape=jax.ShapeDtypeStruct((N, C), jnp.float32),
    )(x, agg, cnt, w_self, w_nbr, g.reshape(1, C), b.reshape(1, C))


def kernel(x_user, x_item, W_self_user, W_self_item, W_nbr_u2i, W_nbr_i2u,
           ln_g_user, ln_b_user, ln_g_item, ln_b_item,
           edge_index_user_to_item, edge_index_item_to_user):
    su2i = edge_index_user_to_item[0].astype(jnp.int32).reshape(NS, NCH, K)
    du2i = edge_index_user_to_item[1].astype(jnp.int32).reshape(NS, NCH, K)
    si2u = edge_index_item_to_user[0].astype(jnp.int32).reshape(NS, NCH, K)
    di2u = edge_index_item_to_user[1].astype(jnp.int32).reshape(NS, NCH, K)

    cnt_i, cnt_u = _sc_counts(du2i, di2u)

    xu, xi = x_user, x_item
    for l in range(L):
        agg_i, agg_u = _sc_aggregate(xu, xi, su2i, du2i, si2u, di2u)
        xu = _tc_dense(xu, agg_u, cnt_u, W_self_user[l], W_nbr_i2u[l],
                       ln_g_user[l], ln_b_user[l])
        xi = _tc_dense(xi, agg_i, cnt_i, W_self_item[l], W_nbr_u2i[l],
                       ln_g_item[l], ln_b_item[l])
    return xu


# speedup vs baseline: 10.6832x; 1.4967x over previous
import functools

import jax
import jax.numpy as jnp
from jax import lax
from jax.experimental import pallas as pl
from jax.experimental.pallas import tpu as pltpu
from jax.experimental.pallas import tpu_sc as plsc

N = 10000
C = 128
E = 320000
L = 4

NS = 16
LANES = 16

EPT = E // NS
K = 80
NCH = EPT // K
G = 50
NG = NCH // G
RPT = 640
NP = NS * RPT
CCH = C // LANES
ZCH = RPT // K

_mesh = plsc.VectorSubcoreMesh(core_axis_name="c", subcore_axis_name="s")
_params = pltpu.CompilerParams(use_tc_tiling_on_sc=False)


@functools.partial(
    pl.kernel,
    out_type=(
        jax.ShapeDtypeStruct((NP, LANES), jnp.float32),
        jax.ShapeDtypeStruct((NP, LANES), jnp.float32),
    ),
    mesh=_mesh,
    compiler_params=_params,
    scratch_types=dict(
        didx_g=pltpu.VMEM((G, K), jnp.int32),
        ones_v=pltpu.VMEM((K, LANES), jnp.float32),
        zero_v=pltpu.VMEM((K, LANES), jnp.float32),
        cnt_sp=pltpu.VMEM_SHARED((NP, LANES), jnp.float32),
    ),
)
def _sc_counts(du2i, di2u, cnt_i, cnt_u, didx_g, ones_v, zero_v, cnt_sp):
    c = lax.axis_index("c")
    s = lax.axis_index("s")

    zeros16 = jnp.zeros((LANES,), jnp.float32)
    ones16 = jnp.ones((LANES,), jnp.float32)

    def _fill(k, _):
        ones_v[k, :] = ones16
        zero_v[k, :] = zeros16
        return _
    lax.fori_loop(0, K, _fill, 0)

    base_r = s * RPT
    for z in range(ZCH):
        pltpu.sync_copy(zero_v, cnt_sp.at[pl.ds(base_r + z * K, K)])
    plsc.subcore_barrier()

    def _count(dst_hbm):
        def group(g, _):
            pltpu.sync_copy(dst_hbm.at[s, pl.ds(g * G, G)], didx_g)
            for jj in range(G):
                pltpu.sync_copy(ones_v, cnt_sp.at[didx_g.at[jj]], add=True)
            return _
        lax.fori_loop(0, NG, group, 0)

    @pl.when(c == 0)
    def _():
        _count(du2i)

    @pl.when(c == 1)
    def _():
        _count(di2u)

    plsc.subcore_barrier()

    @pl.when(c == 0)
    def _():
        pltpu.sync_copy(cnt_sp.at[pl.ds(base_r, RPT)], cnt_i.at[pl.ds(base_r, RPT)])

    @pl.when(c == 1)
    def _():
        pltpu.sync_copy(cnt_sp.at[pl.ds(base_r, RPT)], cnt_u.at[pl.ds(base_r, RPT)])


@functools.partial(
    pl.kernel,
    out_type=(
        jax.ShapeDtypeStruct((NP, C), jnp.float32),
        jax.ShapeDtypeStruct((NP, C), jnp.float32),
    ),
    mesh=_mesh,
    compiler_params=_params,
    scratch_types=dict(
        sidx_g=pltpu.VMEM((G, K), jnp.int32),
        didx_g=pltpu.VMEM((G, K), jnp.int32),
        rows_a=pltpu.VMEM((K, C), jnp.float32),
        rows_b=pltpu.VMEM((K, C), jnp.float32),
        rows_c=pltpu.VMEM((K, C), jnp.float32),
        sem_g=pltpu.SemaphoreType.DMA,
        sem_s=pltpu.SemaphoreType.DMA,
        acc_sp=pltpu.VMEM_SHARED((NP, C), jnp.float32),
    ),
)
def _sc_aggregate(x_user, x_item, su2i, du2i, si2u, di2u,
                  out_i, out_u,
                  sidx_g, didx_g, rows_a, rows_b, rows_c, sem_g, sem_s, acc_sp):
    c = lax.axis_index("c")
    s = lax.axis_index("s")

    zeros16 = jnp.zeros((LANES,), jnp.float32)

    def _zrow(r, _):
        for j in range(CCH):
            rows_a[r, pl.ds(j * LANES, LANES)] = zeros16
        return _
    lax.fori_loop(0, K, _zrow, 0)

    base_r = s * RPT
    for z in range(ZCH):
        pltpu.sync_copy(rows_a, acc_sp.at[pl.ds(base_r + z * K, K)])
    plsc.subcore_barrier()

    def _wait_gather():
        pltpu.make_async_copy(x_user.at[pl.ds(0, K)], rows_a, sem_g).wait()

    def _wait_scatter():
        pltpu.make_async_copy(x_user.at[pl.ds(0, K)], rows_a, sem_s).wait()

    def _accumulate(x_hbm, src_hbm, dst_hbm):
        rbufs = (rows_a, rows_b, rows_c)
        nbuf = len(rbufs)

        def group(g, _):
            pltpu.sync_copy(src_hbm.at[s, pl.ds(g * G, G)], sidx_g)
            pltpu.sync_copy(dst_hbm.at[s, pl.ds(g * G, G)], didx_g)
            for a in range(nbuf - 1):
                pltpu.async_copy(x_hbm.at[sidx_g.at[a]], rbufs[a], sem_g)
            waited = 0
            for jj in range(G):
                rp = rbufs[jj % nbuf]
                _wait_gather()
                if jj + nbuf - 1 < G:
                    if jj >= 1:
                        _wait_scatter()
                        waited += 1
                    pltpu.async_copy(x_hbm.at[sidx_g.at[jj + nbuf - 1]],
                                     rbufs[(jj + nbuf - 1) % nbuf], sem_g)
                pltpu.async_copy(rp, acc_sp.at[didx_g.at[jj]], sem_s, add=True)
            for _w in range(G - waited):
                _wait_scatter()
            return _
        lax.fori_loop(0, NG, group, 0)

    @pl.when(c == 0)
    def _():
        _accumulate(x_user, su2i, du2i)

    @pl.when(c == 1)
    def _():
        _accumulate(x_item, si2u, di2u)

    plsc.subcore_barrier()

    @pl.when(c == 0)
    def _():
        pltpu.sync_copy(acc_sp.at[pl.ds(base_r, RPT)], out_i.at[pl.ds(base_r, RPT)])

    @pl.when(c == 1)
    def _():
        pltpu.sync_copy(acc_sp.at[pl.ds(base_r, RPT)], out_u.at[pl.ds(base_r, RPT)])


BN = 2000


def _tc_body(x_ref, agg_ref, cnt_ref, ws_ref, wn_ref, g_ref, b_ref, o_ref):
    recip = 1.0 / jnp.maximum(cnt_ref[...][:, 0:1], 1.0)
    h = jnp.dot(x_ref[...], ws_ref[...], preferred_element_type=jnp.float32)
    h = h + jnp.dot(agg_ref[...] * recip, wn_ref[...],
                    preferred_element_type=jnp.float32)
    mu = jnp.mean(h, axis=1, keepdims=True)
    var = jnp.mean((h - mu) ** 2, axis=1, keepdims=True)
    hn = (h - mu) * lax.rsqrt(var + 1e-5) * g_ref[...] + b_ref[...]
    o_ref[...] = jnp.maximum(hn, 0.0)


def _tc_dense(x, agg, cnt, w_self, w_nbr, g, b):
    return pl.pallas_call(
        _tc_body,
        grid=(N // BN,),
        in_specs=[
            pl.BlockSpec((BN, C), lambda i: (i, 0)),
            pl.BlockSpec((BN, C), lambda i: (i, 0)),
            pl.BlockSpec((BN, LANES), lambda i: (i, 0)),
            pl.BlockSpec((C, C), lambda i: (0, 0)),
            pl.BlockSpec((C, C), lambda i: (0, 0)),
            pl.BlockSpec((1, C), lambda i: (0, 0)),
            pl.BlockSpec((1, C), lambda i: (0, 0)),
        ],
        out_specs=pl.BlockSpec((BN, C), lambda i: (i, 0)),
        out_shape=jax.ShapeDtypeStruct((N, C), jnp.float32),
    )(x, agg, cnt, w_self, w_nbr, g.reshape(1, C), b.reshape(1, C))


def kernel(x_user, x_item, W_self_user, W_self_item, W_nbr_u2i, W_nbr_i2u,
           ln_g_user, ln_b_user, ln_g_item, ln_b_item,
           edge_index_user_to_item, edge_index_item_to_user):
    su2i = edge_index_user_to_item[0].astype(jnp.int32).reshape(NS, NCH, K)
    du2i = edge_index_user_to_item[1].astype(jnp.int32).reshape(NS, NCH, K)
    si2u = edge_index_item_to_user[0].astype(jnp.int32).reshape(NS, NCH, K)
    di2u = edge_index_item_to_user[1].astype(jnp.int32).reshape(NS, NCH, K)

    cnt_i, cnt_u = _sc_counts(du2i, di2u)

    xu, xi = x_user, x_item
    for l in range(L):
        agg_i, agg_u = _sc_aggregate(xu, xi, su2i, du2i, si2u, di2u)
        xu = _tc_dense(xu, agg_u, cnt_u, W_self_user[l], W_nbr_i2u[l],
                       ln_g_user[l], ln_b_user[l])
        xi = _tc_dense(xi, agg_i, cnt_i, W_self_item[l], W_nbr_u2i[l],
                       ln_g_item[l], ln_b_item[l])
    return xu


# speedup vs baseline: 10.8793x; 1.0184x over previous
import functools

import jax
import jax.numpy as jnp
from jax import lax
from jax.experimental import pallas as pl
from jax.experimental.pallas import tpu as pltpu
from jax.experimental.pallas import tpu_sc as plsc

N = 10000
C = 128
E = 320000
L = 4

NS = 16
LANES = 16

EPT = E // NS
K = 80
NCH = EPT // K
G = 25
NG = NCH // G
RPT = 640
NP = NS * RPT
CCH = C // LANES
ZCH = RPT // K

_mesh = plsc.VectorSubcoreMesh(core_axis_name="c", subcore_axis_name="s")
_params = pltpu.CompilerParams(use_tc_tiling_on_sc=False)


@functools.partial(
    pl.kernel,
    out_type=(
        jax.ShapeDtypeStruct((NP, LANES), jnp.float32),
        jax.ShapeDtypeStruct((NP, LANES), jnp.float32),
    ),
    mesh=_mesh,
    compiler_params=_params,
    scratch_types=dict(
        didx_g=pltpu.VMEM((G, K), jnp.int32),
        ones_v=pltpu.VMEM((K, LANES), jnp.float32),
        zero_v=pltpu.VMEM((K, LANES), jnp.float32),
        cnt_sp=pltpu.VMEM_SHARED((NP, LANES), jnp.float32),
    ),
)
def _sc_counts(du2i, di2u, cnt_i, cnt_u, didx_g, ones_v, zero_v, cnt_sp):
    c = lax.axis_index("c")
    s = lax.axis_index("s")

    zeros16 = jnp.zeros((LANES,), jnp.float32)
    ones16 = jnp.ones((LANES,), jnp.float32)

    def _fill(k, _):
        ones_v[k, :] = ones16
        zero_v[k, :] = zeros16
        return _
    lax.fori_loop(0, K, _fill, 0)

    base_r = s * RPT
    for z in range(ZCH):
        pltpu.sync_copy(zero_v, cnt_sp.at[pl.ds(base_r + z * K, K)])
    plsc.subcore_barrier()

    def _count(dst_hbm):
        def group(g, _):
            pltpu.sync_copy(dst_hbm.at[s, pl.ds(g * G, G)], didx_g)
            for jj in range(G):
                pltpu.sync_copy(ones_v, cnt_sp.at[didx_g.at[jj]], add=True)
            return _
        lax.fori_loop(0, NG, group, 0)

    @pl.when(c == 0)
    def _():
        _count(du2i)

    @pl.when(c == 1)
    def _():
        _count(di2u)

    plsc.subcore_barrier()

    @pl.when(c == 0)
    def _():
        pltpu.sync_copy(cnt_sp.at[pl.ds(base_r, RPT)], cnt_i.at[pl.ds(base_r, RPT)])

    @pl.when(c == 1)
    def _():
        pltpu.sync_copy(cnt_sp.at[pl.ds(base_r, RPT)], cnt_u.at[pl.ds(base_r, RPT)])


@functools.partial(
    pl.kernel,
    out_type=(
        jax.ShapeDtypeStruct((NP, C), jnp.float32),
        jax.ShapeDtypeStruct((NP, C), jnp.float32),
    ),
    mesh=_mesh,
    compiler_params=_params,
    scratch_types=dict(
        sidx_g=pltpu.VMEM((G, K), jnp.int32),
        didx_g=pltpu.VMEM((G, K), jnp.int32),
        rows_a=pltpu.VMEM((K, C), jnp.float32),
        rows_b=pltpu.VMEM((K, C), jnp.float32),
        rows_c=pltpu.VMEM((K, C), jnp.float32),
        rows_d=pltpu.VMEM((K, C), jnp.float32),
        sem_g=pltpu.SemaphoreType.DMA,
        sem_s=pltpu.SemaphoreType.DMA,
        acc_sp=pltpu.VMEM_SHARED((NP, C), jnp.float32),
    ),
)
def _sc_aggregate(x_user, x_item, su2i, du2i, si2u, di2u,
                  out_i, out_u,
                  sidx_g, didx_g, rows_a, rows_b, rows_c, rows_d, sem_g, sem_s, acc_sp):
    c = lax.axis_index("c")
    s = lax.axis_index("s")

    zeros16 = jnp.zeros((LANES,), jnp.float32)

    def _zrow(r, _):
        for j in range(CCH):
            rows_a[r, pl.ds(j * LANES, LANES)] = zeros16
        return _
    lax.fori_loop(0, K, _zrow, 0)

    base_r = s * RPT
    for z in range(ZCH):
        pltpu.sync_copy(rows_a, acc_sp.at[pl.ds(base_r + z * K, K)])
    plsc.subcore_barrier()

    def _wait_gather():
        pltpu.make_async_copy(x_user.at[pl.ds(0, K)], rows_a, sem_g).wait()

    def _wait_scatter():
        pltpu.make_async_copy(x_user.at[pl.ds(0, K)], rows_a, sem_s).wait()

    def _accumulate(x_hbm, src_hbm, dst_hbm):
        rbufs = (rows_a, rows_b, rows_c, rows_d)
        nbuf = len(rbufs)

        def group(g, _):
            pltpu.sync_copy(src_hbm.at[s, pl.ds(g * G, G)], sidx_g)
            pltpu.sync_copy(dst_hbm.at[s, pl.ds(g * G, G)], didx_g)
            for a in range(nbuf - 1):
                pltpu.async_copy(x_hbm.at[sidx_g.at[a]], rbufs[a], sem_g)
            waited = 0
            for jj in range(G):
                rp = rbufs[jj % nbuf]
                _wait_gather()
                if jj + nbuf - 1 < G:
                    if jj >= 1:
                        _wait_scatter()
                        waited += 1
                    pltpu.async_copy(x_hbm.at[sidx_g.at[jj + nbuf - 1]],
                                     rbufs[(jj + nbuf - 1) % nbuf], sem_g)
                pltpu.async_copy(rp, acc_sp.at[didx_g.at[jj]], sem_s, add=True)
            for _w in range(G - waited):
                _wait_scatter()
            return _
        lax.fori_loop(0, NG, group, 0)

    @pl.when(c == 0)
    def _():
        _accumulate(x_user, su2i, du2i)

    @pl.when(c == 1)
    def _():
        _accumulate(x_item, si2u, di2u)

    plsc.subcore_barrier()

    @pl.when(c == 0)
    def _():
        pltpu.sync_copy(acc_sp.at[pl.ds(base_r, RPT)], out_i.at[pl.ds(base_r, RPT)])

    @pl.when(c == 1)
    def _():
        pltpu.sync_copy(acc_sp.at[pl.ds(base_r, RPT)], out_u.at[pl.ds(base_r, RPT)])


BN = 2000


def _tc_body(x_ref, agg_ref, cnt_ref, ws_ref, wn_ref, g_ref, b_ref, o_ref):
    recip = 1.0 / jnp.maximum(cnt_ref[...][:, 0:1], 1.0)
    h = jnp.dot(x_ref[...], ws_ref[...], preferred_element_type=jnp.float32)
    h = h + jnp.dot(agg_ref[...] * recip, wn_ref[...],
                    preferred_element_type=jnp.float32)
    mu = jnp.mean(h, axis=1, keepdims=True)
    var = jnp.mean((h - mu) ** 2, axis=1, keepdims=True)
    hn = (h - mu) * lax.rsqrt(var + 1e-5) * g_ref[...] + b_ref[...]
    o_ref[...] = jnp.maximum(hn, 0.0)


def _tc_dense(x, agg, cnt, w_self, w_nbr, g, b):
    return pl.pallas_call(
        _tc_body,
        grid=(N // BN,),
        in_specs=[
            pl.BlockSpec((BN, C), lambda i: (i, 0)),
            pl.BlockSpec((BN, C), lambda i: (i, 0)),
            pl.BlockSpec((BN, LANES), lambda i: (i, 0)),
            pl.BlockSpec((C, C), lambda i: (0, 0)),
            pl.BlockSpec((C, C), lambda i: (0, 0)),
            pl.BlockSpec((1, C), lambda i: (0, 0)),
            pl.BlockSpec((1, C), lambda i: (0, 0)),
        ],
        out_specs=pl.BlockSpec((BN, C), lambda i: (i, 0)),
        out_shape=jax.ShapeDtypeStruct((N, C), jnp.float32),
    )(x, agg, cnt, w_self, w_nbr, g.reshape(1, C), b.reshape(1, C))


def kernel(x_user, x_item, W_self_user, W_self_item, W_nbr_u2i, W_nbr_i2u,
           ln_g_user, ln_b_user, ln_g_item, ln_b_item,
           edge_index_user_to_item, edge_index_item_to_user):
    su2i = edge_index_user_to_item[0].astype(jnp.int32).reshape(NS, NCH, K)
    du2i = edge_index_user_to_item[1].astype(jnp.int32).reshape(NS, NCH, K)
    si2u = edge_index_item_to_user[0].astype(jnp.int32).reshape(NS, NCH, K)
    di2u = edge_index_item_to_user[1].astype(jnp.int32).reshape(NS, NCH, K)

    cnt_i, cnt_u = _sc_counts(du2i, di2u)

    xu, xi = x_user, x_item
    for l in range(L):
        agg_i, agg_u = _sc_aggregate(xu, xi, su2i, du2i, si2u, di2u)
        xu = _tc_dense(xu, agg_u, cnt_u, W_self_user[l], W_nbr_i2u[l],
                       ln_g_user[l], ln_b_user[l])
        xi = _tc_dense(xi, agg_i, cnt_i, W_self_item[l], W_nbr_u2i[l],
                       ln_g_item[l], ln_b_item[l])
    return xu


# speedup vs baseline: 10.9022x; 1.0021x over previous
import jax
import jax.numpy as jnp
from jax import lax
from jax.experimental import pallas as pl
from jax.experimental.pallas import tpu as pltpu
from jax.experimental.pallas import tpu_sc as plsc

N = 10000
C = 128
E = 320000
L = 4

NS = 16
LANES = 16

EPT = E // NS
K = 80
NCH = EPT // K
G = 25
NG = NCH // G
RPT = 640
NP = NS * RPT
CCH = C // LANES
ZCH = RPT // K
NBUF = 4
G0 = 10

_mesh = plsc.VectorSubcoreMesh(core_axis_name="c", subcore_axis_name="s")
_params = pltpu.CompilerParams(use_tc_tiling_on_sc=False)

_AGG_SCRATCH = dict(
    sidx_g=pltpu.VMEM((G, K), jnp.int32),
    didx_g=pltpu.VMEM((G, K), jnp.int32),
    rows_a=pltpu.VMEM((K, C), jnp.float32),
    rows_b=pltpu.VMEM((K, C), jnp.float32),
    rows_c=pltpu.VMEM((K, C), jnp.float32),
    rows_d=pltpu.VMEM((K, C), jnp.float32),
    sem_g=pltpu.SemaphoreType.DMA,
    sem_s=pltpu.SemaphoreType.DMA,
    acc_sp=pltpu.VMEM_SHARED((NP, C), jnp.float32),
)


def _agg_body(x_user, x_item, su2i, du2i, si2u, di2u, out_i, out_u,
              sidx_g, didx_g, rbufs, sem_g, sem_s, acc_sp,
              counts=None, gsz=G):
    c = lax.axis_index("c")
    s = lax.axis_index("s")
    rows_a = rbufs[0]

    zeros16 = jnp.zeros((LANES,), jnp.float32)

    def _zrow(r, _):
        for j in range(CCH):
            rows_a[r, pl.ds(j * LANES, LANES)] = zeros16
        return _
    lax.fori_loop(0, K, _zrow, 0)

    base_r = s * RPT
    for z in range(ZCH):
        pltpu.sync_copy(rows_a, acc_sp.at[pl.ds(base_r + z * K, K)])

    if counts is not None:
        cnt_i, cnt_u, ones_v, zerol_v, sem_c, cnt_sp = counts
        ones16 = jnp.ones((LANES,), jnp.float32)

        def _fill(r, _):
            ones_v[r, :] = ones16
            zerol_v[r, :] = zeros16
            return _
        lax.fori_loop(0, K, _fill, 0)
        for z in range(ZCH):
            pltpu.sync_copy(zerol_v, cnt_sp.at[pl.ds(base_r + z * K, K)])
    plsc.subcore_barrier()

    def _wait(sem):
        pltpu.make_async_copy(x_user.at[pl.ds(0, K)], rows_a, sem).wait()

    def _accumulate(x_hbm, src_hbm, dst_hbm, cnt_ref):
        nbuf = len(rbufs)

        def group(g, _):
            pltpu.sync_copy(src_hbm.at[s, pl.ds(g * gsz, gsz)], sidx_g)
            pltpu.sync_copy(dst_hbm.at[s, pl.ds(g * gsz, gsz)], didx_g)
            for a in range(nbuf - 1):
                pltpu.async_copy(x_hbm.at[sidx_g.at[a]], rbufs[a], sem_g)
            waited = 0
            for jj in range(gsz):
                rp = rbufs[jj % nbuf]
                _wait(sem_g)
                if jj + nbuf - 1 < gsz:
                    if jj >= 1:
                        _wait(sem_s)
                        waited += 1
                    pltpu.async_copy(x_hbm.at[sidx_g.at[jj + nbuf - 1]],
                                     rbufs[(jj + nbuf - 1) % nbuf], sem_g)
                pltpu.async_copy(rp, acc_sp.at[didx_g.at[jj]], sem_s, add=True)
                if counts is not None:
                    pltpu.async_copy(ones_v, cnt_sp.at[didx_g.at[jj]], sem_c,
                                     add=True)
            for _w in range(gsz - waited):
                _wait(sem_s)
            if counts is not None:
                for _w in range(gsz):
                    pltpu.make_async_copy(cnt_ref.at[pl.ds(0, K)], ones_v,
                                          sem_c).wait()
            return _
        lax.fori_loop(0, NCH // gsz, group, 0)

    @pl.when(c == 0)
    def _():
        _accumulate(x_user, su2i, du2i,
                    counts[0] if counts is not None else None)

    @pl.when(c == 1)
    def _():
        _accumulate(x_item, si2u, di2u,
                    counts[1] if counts is not None else None)

    plsc.subcore_barrier()

    @pl.when(c == 0)
    def _():
        pltpu.sync_copy(acc_sp.at[pl.ds(base_r, RPT)],
                        out_i.at[pl.ds(base_r, RPT)])
        if counts is not None:
            pltpu.sync_copy(cnt_sp.at[pl.ds(base_r, RPT)],
                            cnt_i.at[pl.ds(base_r, RPT)])

    @pl.when(c == 1)
    def _():
        pltpu.sync_copy(acc_sp.at[pl.ds(base_r, RPT)],
                        out_u.at[pl.ds(base_r, RPT)])
        if counts is not None:
            pltpu.sync_copy(cnt_sp.at[pl.ds(base_r, RPT)],
                            cnt_u.at[pl.ds(base_r, RPT)])


@pl.kernel(
    out_type=(
        jax.ShapeDtypeStruct((NP, C), jnp.float32),
        jax.ShapeDtypeStruct((NP, C), jnp.float32),
    ),
    mesh=_mesh,
    compiler_params=_params,
    scratch_types=dict(_AGG_SCRATCH),
)
def _sc_aggregate(x_user, x_item, su2i, du2i, si2u, di2u, out_i, out_u,
                  sidx_g, didx_g, rows_a, rows_b, rows_c, rows_d,
                  sem_g, sem_s, acc_sp):
    _agg_body(x_user, x_item, su2i, du2i, si2u, di2u, out_i, out_u,
              sidx_g, didx_g, (rows_a, rows_b, rows_c, rows_d),
              sem_g, sem_s, acc_sp)


@pl.kernel(
    out_type=(
        jax.ShapeDtypeStruct((NP, C), jnp.float32),
        jax.ShapeDtypeStruct((NP, C), jnp.float32),
        jax.ShapeDtypeStruct((NP, LANES), jnp.float32),
        jax.ShapeDtypeStruct((NP, LANES), jnp.float32),
    ),
    mesh=_mesh,
    compiler_params=_params,
    scratch_types=dict(
        sidx_g=pltpu.VMEM((G0, K), jnp.int32),
        didx_g=pltpu.VMEM((G0, K), jnp.int32),
        rows_a=pltpu.VMEM((K, C), jnp.float32),
        rows_b=pltpu.VMEM((K, C), jnp.float32),
        rows_c=pltpu.VMEM((K, C), jnp.float32),
        sem_g=pltpu.SemaphoreType.DMA,
        sem_s=pltpu.SemaphoreType.DMA,
        acc_sp=pltpu.VMEM_SHARED((NP, C), jnp.float32),
        ones_v=pltpu.VMEM((K, LANES), jnp.float32),
        zerol_v=pltpu.VMEM((K, LANES), jnp.float32),
        sem_c=pltpu.SemaphoreType.DMA,
        cnt_sp=pltpu.VMEM_SHARED((NP, LANES), jnp.float32),
    ),
)
def _sc_aggregate_counts(x_user, x_item, su2i, du2i, si2u, di2u,
                         out_i, out_u, cnt_i, cnt_u,
                         sidx_g, didx_g, rows_a, rows_b, rows_c,
                         sem_g, sem_s, acc_sp,
                         ones_v, zerol_v, sem_c, cnt_sp):
    _agg_body(x_user, x_item, su2i, du2i, si2u, di2u, out_i, out_u,
              sidx_g, didx_g, (rows_a, rows_b, rows_c),
              sem_g, sem_s, acc_sp,
              counts=(cnt_i, cnt_u, ones_v, zerol_v, sem_c, cnt_sp),
              gsz=G0)


BN = 2000


def _tc_self_body(xu_ref, xi_ref, wsu_ref, wsi_ref, hu_ref, hi_ref):
    hu_ref[...] = jnp.dot(xu_ref[...], wsu_ref[...],
                          preferred_element_type=jnp.float32)
    hi_ref[...] = jnp.dot(xi_ref[...], wsi_ref[...],
                          preferred_element_type=jnp.float32)


def _tc_self(xu, xi, wsu, wsi):
    blk = lambda i: (i, 0)
    full = lambda i: (0, 0)
    return pl.pallas_call(
        _tc_self_body,
        grid=(N // BN,),
        in_specs=[
            pl.BlockSpec((BN, C), blk),
            pl.BlockSpec((BN, C), blk),
            pl.BlockSpec((C, C), full),
            pl.BlockSpec((C, C), full),
        ],
        out_specs=[pl.BlockSpec((BN, C), blk), pl.BlockSpec((BN, C), blk)],
        out_shape=[jax.ShapeDtypeStruct((N, C), jnp.float32)] * 2,
    )(xu, xi, wsu, wsi)


def _tc_post_body(hu_ref, aggu_ref, cntu_ref, wnu_ref, gu_ref, bu_ref,
                  hi_ref, aggi_ref, cnti_ref, wni_ref, gi_ref, bi_ref,
                  ou_ref, oi_ref):
    def side(h_ref, agg_ref, cnt_ref, wn_ref, g_ref, b_ref, o_ref):
        recip = 1.0 / jnp.maximum(cnt_ref[...][:, 0:1], 1.0)
        h = h_ref[...] + jnp.dot(agg_ref[...] * recip, wn_ref[...],
                                 preferred_element_type=jnp.float32)
        mu = jnp.mean(h, axis=1, keepdims=True)
        var = jnp.mean((h - mu) ** 2, axis=1, keepdims=True)
        hn = (h - mu) * lax.rsqrt(var + 1e-5) * g_ref[...] + b_ref[...]
        o_ref[...] = jnp.maximum(hn, 0.0)

    side(hu_ref, aggu_ref, cntu_ref, wnu_ref, gu_ref, bu_ref, ou_ref)
    side(hi_ref, aggi_ref, cnti_ref, wni_ref, gi_ref, bi_ref, oi_ref)


def _tc_post(hu, aggu, cntu, wnu, gu, bu, hi, aggi, cnti, wni, gi, bi):
    blk = lambda i: (i, 0)
    full = lambda i: (0, 0)
    side_specs = [
        pl.BlockSpec((BN, C), blk),
        pl.BlockSpec((BN, C), blk),
        pl.BlockSpec((BN, LANES), blk),
        pl.BlockSpec((C, C), full),
        pl.BlockSpec((1, C), full),
        pl.BlockSpec((1, C), full),
    ]
    return pl.pallas_call(
        _tc_post_body,
        grid=(N // BN,),
        in_specs=side_specs + side_specs,
        out_specs=[pl.BlockSpec((BN, C), blk), pl.BlockSpec((BN, C), blk)],
        out_shape=[jax.ShapeDtypeStruct((N, C), jnp.float32)] * 2,
    )(hu, aggu, cntu, wnu, gu.reshape(1, C), bu.reshape(1, C),
      hi, aggi, cnti, wni, gi.reshape(1, C), bi.reshape(1, C))


def kernel(x_user, x_item, W_self_user, W_self_item, W_nbr_u2i, W_nbr_i2u,
           ln_g_user, ln_b_user, ln_g_item, ln_b_item,
           edge_index_user_to_item, edge_index_item_to_user):
    su2i = edge_index_user_to_item[0].astype(jnp.int32).reshape(NS, NCH, K)
    du2i = edge_index_user_to_item[1].astype(jnp.int32).reshape(NS, NCH, K)
    si2u = edge_index_item_to_user[0].astype(jnp.int32).reshape(NS, NCH, K)
    di2u = edge_index_item_to_user[1].astype(jnp.int32).reshape(NS, NCH, K)

    xu, xi = x_user, x_item
    cnt_i = cnt_u = None
    for l in range(L):
        if l == 0:
            agg_i, agg_u, cnt_i, cnt_u = _sc_aggregate_counts(
                xu, xi, su2i, du2i, si2u, di2u)
        else:
            agg_i, agg_u = _sc_aggregate(xu, xi, su2i, du2i, si2u, di2u)
        hu, hi = _tc_self(xu, xi, W_self_user[l], W_self_item[l])
        xu, xi = _tc_post(hu, agg_u, cnt_u, W_nbr_i2u[l],
                          ln_g_user[l], ln_b_user[l],
                          hi, agg_i, cnt_i, W_nbr_u2i[l],
                          ln_g_item[l], ln_b_item[l])
    return xu


# speedup vs baseline: 10.9074x; 1.0005x over previous
import jax
import jax.numpy as jnp
from jax import lax
from jax.experimental import pallas as pl
from jax.experimental.pallas import tpu as pltpu
from jax.experimental.pallas import tpu_sc as plsc

N = 10000
C = 128
E = 320000
L = 4

NS = 16
LANES = 16

EPT = E // NS
K = 80
NCH = EPT // K
G = 25
NG = NCH // G
RPT = 640
NP = NS * RPT
CCH = C // LANES
ZCH = RPT // K
NBUF = 4
G0 = 10

_mesh = plsc.VectorSubcoreMesh(core_axis_name="c", subcore_axis_name="s")
_params = pltpu.CompilerParams(use_tc_tiling_on_sc=False)

_AGG_SCRATCH = dict(
    sidx_g=pltpu.VMEM((G, K), jnp.int32),
    didx_g=pltpu.VMEM((G, K), jnp.int32),
    rows_a=pltpu.VMEM((K, C), jnp.float32),
    rows_b=pltpu.VMEM((K, C), jnp.float32),
    rows_c=pltpu.VMEM((K, C), jnp.float32),
    rows_d=pltpu.VMEM((K, C), jnp.float32),
    sem_g=pltpu.SemaphoreType.DMA,
    sem_s=pltpu.SemaphoreType.DMA,
    acc_sp=pltpu.VMEM_SHARED((NP, C), jnp.float32),
)


def _agg_body(x_user, x_item, su2i, du2i, si2u, di2u, out_i, out_u,
              sidx_g, didx_g, rbufs, sem_g, sem_s, acc_sp,
              counts=None, gsz=G):
    c = lax.axis_index("c")
    s = lax.axis_index("s")
    rows_a = rbufs[0]

    zeros16 = jnp.zeros((LANES,), jnp.float32)

    def _zrow(r, _):
        for j in range(CCH):
            rows_a[r, pl.ds(j * LANES, LANES)] = zeros16
        return _
    lax.fori_loop(0, K, _zrow, 0)

    base_r = s * RPT
    for z in range(ZCH):
        pltpu.sync_copy(rows_a, acc_sp.at[pl.ds(base_r + z * K, K)])

    if counts is not None:
        cnt_i, cnt_u, ones_v, zerol_v, sem_c, cnt_sp = counts
        ones16 = jnp.ones((LANES,), jnp.float32)

        def _fill(r, _):
            ones_v[r, :] = ones16
            zerol_v[r, :] = zeros16
            return _
        lax.fori_loop(0, K, _fill, 0)
        for z in range(ZCH):
            pltpu.sync_copy(zerol_v, cnt_sp.at[pl.ds(base_r + z * K, K)])
    plsc.subcore_barrier()

    def _wait(sem):
        pltpu.make_async_copy(x_user.at[pl.ds(0, K)], rows_a, sem).wait()

    def _accumulate(x_hbm, src_hbm, dst_hbm, cnt_ref):
        nbuf = len(rbufs)

        def group(g, _):
            pltpu.sync_copy(src_hbm.at[s, pl.ds(g * gsz, gsz)], sidx_g)
            pltpu.sync_copy(dst_hbm.at[s, pl.ds(g * gsz, gsz)], didx_g)
            for a in range(nbuf - 1):
                pltpu.async_copy(x_hbm.at[sidx_g.at[a]], rbufs[a], sem_g)
            waited = 0
            for jj in range(gsz):
                rp = rbufs[jj % nbuf]
                _wait(sem_g)
                if jj + nbuf - 1 < gsz:
                    if jj >= 1:
                        _wait(sem_s)
                        waited += 1
                    pltpu.async_copy(x_hbm.at[sidx_g.at[jj + nbuf - 1]],
                                     rbufs[(jj + nbuf - 1) % nbuf], sem_g)
                pltpu.async_copy(rp, acc_sp.at[didx_g.at[jj]], sem_s, add=True)
                if counts is not None:
                    pltpu.async_copy(ones_v, cnt_sp.at[didx_g.at[jj]], sem_c,
                                     add=True)
            for _w in range(gsz - waited):
                _wait(sem_s)
            if counts is not None:
                for _w in range(gsz):
                    pltpu.make_async_copy(cnt_ref.at[pl.ds(0, K)], ones_v,
                                          sem_c).wait()
            return _
        lax.fori_loop(0, NCH // gsz, group, 0)

    @pl.when(c == 0)
    def _():
        _accumulate(x_user, su2i, du2i,
                    counts[0] if counts is not None else None)

    @pl.when(c == 1)
    def _():
        _accumulate(x_item, si2u, di2u,
                    counts[1] if counts is not None else None)

    plsc.subcore_barrier()

    @pl.when(c == 0)
    def _():
        pltpu.sync_copy(acc_sp.at[pl.ds(base_r, RPT)],
                        out_i.at[pl.ds(base_r, RPT)])
        if counts is not None:
            pltpu.sync_copy(cnt_sp.at[pl.ds(base_r, RPT)],
                            cnt_i.at[pl.ds(base_r, RPT)])

    @pl.when(c == 1)
    def _():
        pltpu.sync_copy(acc_sp.at[pl.ds(base_r, RPT)],
                        out_u.at[pl.ds(base_r, RPT)])
        if counts is not None:
            pltpu.sync_copy(cnt_sp.at[pl.ds(base_r, RPT)],
                            cnt_u.at[pl.ds(base_r, RPT)])


@pl.kernel(
    out_type=(
        jax.ShapeDtypeStruct((NP, C), jnp.float32),
        jax.ShapeDtypeStruct((NP, C), jnp.float32),
    ),
    mesh=_mesh,
    compiler_params=_params,
    scratch_types=dict(_AGG_SCRATCH),
)
def _sc_aggregate(x_user, x_item, su2i, du2i, si2u, di2u, out_i, out_u,
                  sidx_g, didx_g, rows_a, rows_b, rows_c, rows_d,
                  sem_g, sem_s, acc_sp):
    _agg_body(x_user, x_item, su2i, du2i, si2u, di2u, out_i, out_u,
              sidx_g, didx_g, (rows_a, rows_b, rows_c, rows_d),
              sem_g, sem_s, acc_sp)


@pl.kernel(
    out_type=(
        jax.ShapeDtypeStruct((NP, C), jnp.float32),
        jax.ShapeDtypeStruct((NP, C), jnp.float32),
        jax.ShapeDtypeStruct((NP, LANES), jnp.float32),
        jax.ShapeDtypeStruct((NP, LANES), jnp.float32),
    ),
    mesh=_mesh,
    compiler_params=_params,
    scratch_types=dict(
        sidx_g=pltpu.VMEM((G0, K), jnp.int32),
        didx_g=pltpu.VMEM((G0, K), jnp.int32),
        rows_a=pltpu.VMEM((K, C), jnp.float32),
        rows_b=pltpu.VMEM((K, C), jnp.float32),
        rows_c=pltpu.VMEM((K, C), jnp.float32),
        sem_g=pltpu.SemaphoreType.DMA,
        sem_s=pltpu.SemaphoreType.DMA,
        acc_sp=pltpu.VMEM_SHARED((NP, C), jnp.float32),
        ones_v=pltpu.VMEM((K, LANES), jnp.float32),
        zerol_v=pltpu.VMEM((K, LANES), jnp.float32),
        sem_c=pltpu.SemaphoreType.DMA,
        cnt_sp=pltpu.VMEM_SHARED((NP, LANES), jnp.float32),
    ),
)
def _sc_aggregate_counts(x_user, x_item, su2i, du2i, si2u, di2u,
                         out_i, out_u, cnt_i, cnt_u,
                         sidx_g, didx_g, rows_a, rows_b, rows_c,
                         sem_g, sem_s, acc_sp,
                         ones_v, zerol_v, sem_c, cnt_sp):
    _agg_body(x_user, x_item, su2i, du2i, si2u, di2u, out_i, out_u,
              sidx_g, didx_g, (rows_a, rows_b, rows_c),
              sem_g, sem_s, acc_sp,
              counts=(cnt_i, cnt_u, ones_v, zerol_v, sem_c, cnt_sp),
              gsz=G0)


BN = 2000


def _tc_dense_body(xu_ref, aggu_ref, cntu_ref, wsu_ref, wnu_ref, gu_ref,
                   bu_ref, xi_ref, aggi_ref, cnti_ref, wsi_ref, wni_ref,
                   gi_ref, bi_ref, ou_ref, oi_ref):
    def side(x_ref, agg_ref, cnt_ref, ws_ref, wn_ref, g_ref, b_ref, o_ref):
        recip = 1.0 / jnp.maximum(cnt_ref[...][:, 0:1], 1.0)
        h = jnp.dot(x_ref[...], ws_ref[...],
                    preferred_element_type=jnp.float32)
        h = h + jnp.dot(agg_ref[...] * recip, wn_ref[...],
                        preferred_element_type=jnp.float32)
        mu = jnp.mean(h, axis=1, keepdims=True)
        var = jnp.mean((h - mu) ** 2, axis=1, keepdims=True)
        hn = (h - mu) * lax.rsqrt(var + 1e-5) * g_ref[...] + b_ref[...]
        o_ref[...] = jnp.maximum(hn, 0.0)

    side(xu_ref, aggu_ref, cntu_ref, wsu_ref, wnu_ref, gu_ref, bu_ref, ou_ref)
    side(xi_ref, aggi_ref, cnti_ref, wsi_ref, wni_ref, gi_ref, bi_ref, oi_ref)


def _tc_dense(xu, aggu, cntu, wsu, wnu, gu, bu,
              xi, aggi, cnti, wsi, wni, gi, bi):
    blk = lambda i: (i, 0)
    full = lambda i: (0, 0)
    side_specs = [
        pl.BlockSpec((BN, C), blk),
        pl.BlockSpec((BN, C), blk),
        pl.BlockSpec((BN, LANES), blk),
        pl.BlockSpec((C, C), full),
        pl.BlockSpec((C, C), full),
        pl.BlockSpec((1, C), full),
        pl.BlockSpec((1, C), full),
    ]
    return pl.pallas_call(
        _tc_dense_body,
        grid=(N // BN,),
        in_specs=side_specs + side_specs,
        out_specs=[pl.BlockSpec((BN, C), blk), pl.BlockSpec((BN, C), blk)],
        out_shape=[jax.ShapeDtypeStruct((N, C), jnp.float32)] * 2,
    )(xu, aggu, cntu, wsu, wnu, gu.reshape(1, C), bu.reshape(1, C),
      xi, aggi, cnti, wsi, wni, gi.reshape(1, C), bi.reshape(1, C))


def kernel(x_user, x_item, W_self_user, W_self_item, W_nbr_u2i, W_nbr_i2u,
           ln_g_user, ln_b_user, ln_g_item, ln_b_item,
           edge_index_user_to_item, edge_index_item_to_user):
    su2i = edge_index_user_to_item[0].astype(jnp.int32).reshape(NS, NCH, K)
    du2i = edge_index_user_to_item[1].astype(jnp.int32).reshape(NS, NCH, K)
    si2u = edge_index_item_to_user[0].astype(jnp.int32).reshape(NS, NCH, K)
    di2u = edge_index_item_to_user[1].astype(jnp.int32).reshape(NS, NCH, K)

    xu, xi = x_user, x_item
    cnt_i = cnt_u = None
    for l in range(L):
        if l == 0:
            agg_i, agg_u, cnt_i, cnt_u = _sc_aggregate_counts(
                xu, xi, su2i, du2i, si2u, di2u)
        else:
            agg_i, agg_u = _sc_aggregate(xu, xi, su2i, du2i, si2u, di2u)
        xu, xi = _tc_dense(xu, agg_u, cnt_u, W_self_user[l], W_nbr_i2u[l],
                           ln_g_user[l], ln_b_user[l],
                           xi, agg_i, cnt_i, W_self_item[l], W_nbr_u2i[l],
                           ln_g_item[l], ln_b_item[l])
    return xu


# speedup vs baseline: 11.2071x; 1.0275x over previous
import jax
import jax.numpy as jnp
from jax import lax
from jax.experimental import pallas as pl
from jax.experimental.pallas import tpu as pltpu
from jax.experimental.pallas import tpu_sc as plsc

N = 10000
C = 128
E = 320000
L = 4

NS = 16
LANES = 16

EPT = E // NS
K = 80
NCH = EPT // K
G = 25
NG = NCH // G
RPT = 640
NP = NS * RPT
CCH = C // LANES
ZCH = RPT // K
NBUF = 4
G0 = 25

_mesh = plsc.VectorSubcoreMesh(core_axis_name="c", subcore_axis_name="s")
_params = pltpu.CompilerParams(use_tc_tiling_on_sc=False)

_AGG_SCRATCH = dict(
    sidx_g=pltpu.VMEM((G, K), jnp.int32),
    didx_g=pltpu.VMEM((G, K), jnp.int32),
    rows_a=pltpu.VMEM((K, C), jnp.float32),
    rows_b=pltpu.VMEM((K, C), jnp.float32),
    rows_c=pltpu.VMEM((K, C), jnp.float32),
    rows_d=pltpu.VMEM((K, C), jnp.float32),
    sem_g=pltpu.SemaphoreType.DMA,
    sem_s=pltpu.SemaphoreType.DMA,
    acc_sp=pltpu.VMEM_SHARED((NP, C), jnp.float32),
)


def _agg_body(x_user, x_item, su2i, du2i, si2u, di2u, out_i, out_u,
              sidx_g, didx_g, rbufs, sem_g, sem_s, acc_sp,
              counts=None, gsz=G):
    c = lax.axis_index("c")
    s = lax.axis_index("s")
    rows_a = rbufs[0]

    zeros16 = jnp.zeros((LANES,), jnp.float32)

    def _zrow(r, _):
        for j in range(CCH):
            rows_a[r, pl.ds(j * LANES, LANES)] = zeros16
        return _
    lax.fori_loop(0, K, _zrow, 0)

    base_r = s * RPT
    for z in range(ZCH):
        pltpu.sync_copy(rows_a, acc_sp.at[pl.ds(base_r + z * K, K)])

    if counts is not None:
        cnt_i, cnt_u, ones_v, zerol_v, sem_c, cnt_sp = counts
        ones16 = jnp.ones((LANES,), jnp.float32)

        def _fill(r, _):
            ones_v[r, :] = ones16
            zerol_v[r, :] = zeros16
            return _
        lax.fori_loop(0, K, _fill, 0)
        for z in range(ZCH):
            pltpu.sync_copy(zerol_v, cnt_sp.at[pl.ds(base_r + z * K, K)])
    plsc.subcore_barrier()

    def _wait(sem):
        pltpu.make_async_copy(x_user.at[pl.ds(0, K)], rows_a, sem).wait()

    def _accumulate(x_hbm, src_hbm, dst_hbm, cnt_ref):
        nbuf = len(rbufs)

        def group(g, _):
            pltpu.sync_copy(src_hbm.at[s, pl.ds(g * gsz, gsz)], sidx_g)
            pltpu.sync_copy(dst_hbm.at[s, pl.ds(g * gsz, gsz)], didx_g)
            for a in range(nbuf - 1):
                pltpu.async_copy(x_hbm.at[sidx_g.at[a]], rbufs[a], sem_g)
            waited = 0
            for jj in range(gsz):
                rp = rbufs[jj % nbuf]
                _wait(sem_g)
                if jj + nbuf - 1 < gsz:
                    if jj >= 1:
                        _wait(sem_s)
                        waited += 1
                    pltpu.async_copy(x_hbm.at[sidx_g.at[jj + nbuf - 1]],
                                     rbufs[(jj + nbuf - 1) % nbuf], sem_g)
                pltpu.async_copy(rp, acc_sp.at[didx_g.at[jj]], sem_s, add=True)
                if counts is not None:
                    pltpu.async_copy(ones_v, cnt_sp.at[didx_g.at[jj]], sem_c,
                                     add=True)
            for _w in range(gsz - waited):
                _wait(sem_s)
            if counts is not None:
                for _w in range(gsz):
                    pltpu.make_async_copy(cnt_ref.at[pl.ds(0, K)], ones_v,
                                          sem_c).wait()
            return _
        lax.fori_loop(0, NCH // gsz, group, 0)

    @pl.when(c == 0)
    def _():
        _accumulate(x_user, su2i, du2i,
                    counts[0] if counts is not None else None)

    @pl.when(c == 1)
    def _():
        _accumulate(x_item, si2u, di2u,
                    counts[1] if counts is not None else None)

    plsc.subcore_barrier()

    @pl.when(c == 0)
    def _():
        pltpu.sync_copy(acc_sp.at[pl.ds(base_r, RPT)],
                        out_i.at[pl.ds(base_r, RPT)])
        if counts is not None:
            pltpu.sync_copy(cnt_sp.at[pl.ds(base_r, RPT)],
                            cnt_i.at[pl.ds(base_r, RPT)])

    @pl.when(c == 1)
    def _():
        pltpu.sync_copy(acc_sp.at[pl.ds(base_r, RPT)],
                        out_u.at[pl.ds(base_r, RPT)])
        if counts is not None:
            pltpu.sync_copy(cnt_sp.at[pl.ds(base_r, RPT)],
                            cnt_u.at[pl.ds(base_r, RPT)])


@pl.kernel(
    out_type=(
        jax.ShapeDtypeStruct((NP, C), jnp.float32),
        jax.ShapeDtypeStruct((NP, C), jnp.float32),
    ),
    mesh=_mesh,
    compiler_params=_params,
    scratch_types=dict(_AGG_SCRATCH),
)
def _sc_aggregate(x_user, x_item, su2i, du2i, si2u, di2u, out_i, out_u,
                  sidx_g, didx_g, rows_a, rows_b, rows_c, rows_d,
                  sem_g, sem_s, acc_sp):
    _agg_body(x_user, x_item, su2i, du2i, si2u, di2u, out_i, out_u,
              sidx_g, didx_g, (rows_a, rows_b, rows_c, rows_d),
              sem_g, sem_s, acc_sp)


@pl.kernel(
    out_type=(
        jax.ShapeDtypeStruct((NP, C), jnp.float32),
        jax.ShapeDtypeStruct((NP, C), jnp.float32),
        jax.ShapeDtypeStruct((NP, LANES), jnp.float32),
        jax.ShapeDtypeStruct((NP, LANES), jnp.float32),
    ),
    mesh=_mesh,
    compiler_params=_params,
    scratch_types=dict(
        sidx_g=pltpu.VMEM((G0, K), jnp.int32),
        didx_g=pltpu.VMEM((G0, K), jnp.int32),
        rows_a=pltpu.VMEM((K, C), jnp.float32),
        rows_b=pltpu.VMEM((K, C), jnp.float32),
        rows_c=pltpu.VMEM((K, C), jnp.float32),
        sem_g=pltpu.SemaphoreType.DMA,
        sem_s=pltpu.SemaphoreType.DMA,
        acc_sp=pltpu.VMEM_SHARED((NP, C), jnp.float32),
        ones_v=pltpu.VMEM((K, LANES), jnp.float32),
        zerol_v=pltpu.VMEM((K, LANES), jnp.float32),
        sem_c=pltpu.SemaphoreType.DMA,
        cnt_sp=pltpu.VMEM_SHARED((NP, LANES), jnp.float32),
    ),
)
def _sc_aggregate_counts(x_user, x_item, su2i, du2i, si2u, di2u,
                         out_i, out_u, cnt_i, cnt_u,
                         sidx_g, didx_g, rows_a, rows_b, rows_c,
                         sem_g, sem_s, acc_sp,
                         ones_v, zerol_v, sem_c, cnt_sp):
    _agg_body(x_user, x_item, su2i, du2i, si2u, di2u, out_i, out_u,
              sidx_g, didx_g, (rows_a, rows_b, rows_c),
              sem_g, sem_s, acc_sp,
              counts=(cnt_i, cnt_u, ones_v, zerol_v, sem_c, cnt_sp),
              gsz=G0)


BN = 2000


def _tc_dense_body(xu_ref, aggu_ref, cntu_ref, wsu_ref, wnu_ref, gu_ref,
                   bu_ref, xi_ref, aggi_ref, cnti_ref, wsi_ref, wni_ref,
                   gi_ref, bi_ref, ou_ref, oi_ref):
    def side(x_ref, agg_ref, cnt_ref, ws_ref, wn_ref, g_ref, b_ref, o_ref):
        recip = 1.0 / jnp.maximum(cnt_ref[...][:, 0:1], 1.0)
        h = jnp.dot(x_ref[...], ws_ref[...],
                    preferred_element_type=jnp.float32)
        h = h + jnp.dot(agg_ref[...] * recip, wn_ref[...],
                        preferred_element_type=jnp.float32)
        mu = jnp.mean(h, axis=1, keepdims=True)
        var = jnp.mean((h - mu) ** 2, axis=1, keepdims=True)
        hn = (h - mu) * lax.rsqrt(var + 1e-5) * g_ref[...] + b_ref[...]
        o_ref[...] = jnp.maximum(hn, 0.0)

    side(xu_ref, aggu_ref, cntu_ref, wsu_ref, wnu_ref, gu_ref, bu_ref, ou_ref)
    side(xi_ref, aggi_ref, cnti_ref, wsi_ref, wni_ref, gi_ref, bi_ref, oi_ref)


def _tc_dense(xu, aggu, cntu, wsu, wnu, gu, bu,
              xi, aggi, cnti, wsi, wni, gi, bi):
    blk = lambda i: (i, 0)
    full = lambda i: (0, 0)
    side_specs = [
        pl.BlockSpec((BN, C), blk),
        pl.BlockSpec((BN, C), blk),
        pl.BlockSpec((BN, LANES), blk),
        pl.BlockSpec((C, C), full),
        pl.BlockSpec((C, C), full),
        pl.BlockSpec((1, C), full),
        pl.BlockSpec((1, C), full),
    ]
    return pl.pallas_call(
        _tc_dense_body,
        grid=(N // BN,),
        in_specs=side_specs + side_specs,
        out_specs=[pl.BlockSpec((BN, C), blk), pl.BlockSpec((BN, C), blk)],
        out_shape=[jax.ShapeDtypeStruct((N, C), jnp.float32)] * 2,
    )(xu, aggu, cntu, wsu, wnu, gu.reshape(1, C), bu.reshape(1, C),
      xi, aggi, cnti, wsi, wni, gi.reshape(1, C), bi.reshape(1, C))


def kernel(x_user, x_item, W_self_user, W_self_item, W_nbr_u2i, W_nbr_i2u,
           ln_g_user, ln_b_user, ln_g_item, ln_b_item,
           edge_index_user_to_item, edge_index_item_to_user):
    su2i = edge_index_user_to_item[0].astype(jnp.int32).reshape(NS, NCH, K)
    du2i = edge_index_user_to_item[1].astype(jnp.int32).reshape(NS, NCH, K)
    si2u = edge_index_item_to_user[0].astype(jnp.int32).reshape(NS, NCH, K)
    di2u = edge_index_item_to_user[1].astype(jnp.int32).reshape(NS, NCH, K)

    xu, xi = x_user, x_item
    cnt_i = cnt_u = None
    for l in range(L):
        if l == 0:
            agg_i, agg_u, cnt_i, cnt_u = _sc_aggregate_counts(
                xu, xi, su2i, du2i, si2u, di2u)
        else:
            agg_i, agg_u = _sc_aggregate(xu, xi, su2i, du2i, si2u, di2u)
        xu, xi = _tc_dense(xu, agg_u, cnt_u, W_self_user[l], W_nbr_i2u[l],
                           ln_g_user[l], ln_b_user[l],
                           xi, agg_i, cnt_i, W_self_item[l], W_nbr_u2i[l],
                           ln_g_item[l], ln_b_item[l])
    return xu
